# Initial kernel scaffold; baseline (speedup 1.0000x reference)
import jax
import jax.numpy as jnp
from jax.experimental import pallas as pl


def kernel(x, edge_index, W1, b1, gamma, beta, W2, b2):
    raise NotImplementedError("write your pallas kernel here")


# speedup vs baseline: 9.2752x; 9.2752x over previous
import functools

import jax
import jax.numpy as jnp
from jax import lax
from jax.experimental import pallas as pl
from jax.experimental.pallas import tpu as pltpu
from jax.experimental.pallas import tpu_sc as plsc

N = 10000
D = 128
DH = D // 2
N_PAD = 10240
NC = 2
NS = 16
NW = NC * NS
CS = 128
CH = 80
EPT = CS * CH
EP = EPT * NW
ROWS_PT = N_PAD // NS

_mesh = plsc.VectorSubcoreMesh(core_axis_name="c", subcore_axis_name="s")


@functools.partial(
    pl.kernel,
    mesh=_mesh,
    out_type=jax.ShapeDtypeStruct((NC, N_PAD), jnp.float32),
    scratch_types=[
        pltpu.VMEM((CH, CS), jnp.int32),
        pltpu.VMEM((CS,), jnp.float32),
        pltpu.VMEM((ROWS_PT,), jnp.float32),
        pltpu.VMEM_SHARED((N_PAD,), jnp.float32),
    ],
)
def _sc_deg(dst_hbm, out_hbm, dstb, ones_v, zbuf, acc):
    cid = lax.axis_index("c")
    sid = lax.axis_index("s")
    wid = sid * NC + cid

    def fill(i, _):
        ones_v[pl.ds(i * 16, 16)] = jnp.ones((16,), jnp.float32)
        return 0

    lax.fori_loop(0, CS // 16, fill, 0)

    def zero(i, _):
        zbuf[pl.ds(i * 16, 16)] = jnp.zeros((16,), jnp.float32)
        return 0

    lax.fori_loop(0, ROWS_PT // 16, zero, 0)
    pltpu.sync_copy(zbuf, acc.at[pl.ds(sid * ROWS_PT, ROWS_PT)])
    plsc.subcore_barrier()

    pltpu.sync_copy(dst_hbm.at[wid], dstb)

    def count(j, _):
        pltpu.sync_copy(ones_v, acc.at[dstb.at[j]], add=True)
        return 0

    lax.fori_loop(0, CH, count, 0)
    plsc.subcore_barrier()
    pltpu.sync_copy(
        acc.at[pl.ds(sid * ROWS_PT, ROWS_PT)],
        out_hbm.at[cid, pl.ds(sid * ROWS_PT, ROWS_PT)],
    )


@functools.partial(
    pl.kernel,
    mesh=_mesh,
    out_type=jax.ShapeDtypeStruct((NC, N_PAD, D), jnp.float32),
    scratch_types=[
        pltpu.VMEM((CH, CS), jnp.int32),
        pltpu.VMEM((CH, CS), jnp.int32),
        pltpu.VMEM((CS, D), jnp.float32),
        pltpu.VMEM_SHARED((N_PAD, D), jnp.float32),
    ],
)
def _sc_scatter(g_hbm, src_hbm, dst_hbm, out_hbm, srcb, dstb, rows_a, acc):
    cid = lax.axis_index("c")
    sid = lax.axis_index("s")
    wid = sid * NC + cid

    def zrow(r, _):
        for cc in range(D // 16):
            rows_a[r, pl.ds(cc * 16, 16)] = jnp.zeros((16,), jnp.float32)
        return 0

    lax.fori_loop(0, CS, zrow, 0)

    def zcopy(k, _):
        pltpu.sync_copy(rows_a, acc.at[pl.ds(sid * ROWS_PT + k * CS, CS)])
        return 0

    lax.fori_loop(0, ROWS_PT // CS, zcopy, 0)
    plsc.subcore_barrier()

    pltpu.sync_copy(src_hbm.at[wid], srcb)
    pltpu.sync_copy(dst_hbm.at[wid], dstb)

    def step(j, _):
        pltpu.sync_copy(g_hbm.at[srcb.at[j]], rows_a)
        pltpu.sync_copy(rows_a, acc.at[dstb.at[j]], add=True)
        return 0

    lax.fori_loop(0, CH, step, 0)
    plsc.subcore_barrier()
    pltpu.sync_copy(
        acc.at[pl.ds(sid * ROWS_PT, ROWS_PT)],
        out_hbm.at[cid, pl.ds(sid * ROWS_PT, ROWS_PT)],
    )


_BLK = 1024
_GRID = N_PAD // _BLK


def _tc_prep_body(dp_ref, x_ref, w_ref, g_ref, dinv_ref):
    deg = dp_ref[0] + dp_ref[1] + 1.0
    dinv = lax.rsqrt(deg)
    dinv_ref[...] = dinv
    h = jnp.dot(x_ref[...], w_ref[...], preferred_element_type=jnp.float32)
    g_ref[...] = dinv * h


def _tc_prep(deg_parts, x_pad, w1):
    return pl.pallas_call(
        _tc_prep_body,
        grid=(_GRID,),
        in_specs=[
            pl.BlockSpec((NC, _BLK, 1), lambda i: (0, i, 0)),
            pl.BlockSpec((_BLK, D), lambda i: (i, 0)),
            pl.BlockSpec((D, D), lambda i: (0, 0)),
        ],
        out_specs=[
            pl.BlockSpec((_BLK, D), lambda i: (i, 0)),
            pl.BlockSpec((_BLK, 1), lambda i: (i, 0)),
        ],
        out_shape=[
            jax.ShapeDtypeStruct((N_PAD, D), jnp.float32),
            jax.ShapeDtypeStruct((N_PAD, 1), jnp.float32),
        ],
    )(deg_parts, x_pad, w1)


def _tc_combine_stats_body(p_ref, g_ref, dinv_ref, b_ref, out_ref, st_ref):
    i = pl.program_id(0)
    o = dinv_ref[...] * (p_ref[0] + p_ref[1] + g_ref[...]) + b_ref[...]
    out_ref[...] = o
    rows = lax.broadcasted_iota(jnp.int32, (_BLK, 1), 0) + i * _BLK
    om = jnp.where(rows < N, o, 0.0)

    @pl.when(i == 0)
    def _():
        st_ref[...] = jnp.zeros_like(st_ref)

    st_ref[0:1, :] += jnp.sum(om, axis=0, keepdims=True)
    st_ref[1:2, :] += jnp.sum(om * om, axis=0, keepdims=True)


_COMBINE_SPECS = [
    pl.BlockSpec((NC, _BLK, D), lambda i: (0, i, 0)),
    pl.BlockSpec((_BLK, D), lambda i: (i, 0)),
    pl.BlockSpec((_BLK, 1), lambda i: (i, 0)),
    pl.BlockSpec((1, D), lambda i: (0, 0)),
]


def _tc_combine_stats(parts, g, dinv, b):
    return pl.pallas_call(
        _tc_combine_stats_body,
        grid=(_GRID,),
        in_specs=_COMBINE_SPECS,
        out_specs=[
            pl.BlockSpec((_BLK, D), lambda i: (i, 0)),
            pl.BlockSpec((8, D), lambda i: (0, 0)),
        ],
        out_shape=[
            jax.ShapeDtypeStruct((N_PAD, D), jnp.float32),
            jax.ShapeDtypeStruct((8, D), jnp.float32),
        ],
    )(parts, g, dinv, b)


def _tc_bn_mm_body(h_ref, st_ref, gam_ref, bet_ref, w_ref, dinv_ref, g_ref):
    mean = st_ref[0:1, :] * (1.0 / N)
    ex2 = st_ref[1:2, :] * (1.0 / N)
    var = ex2 - mean * mean
    scale = gam_ref[...] * lax.rsqrt(var + 1e-5)
    y = jnp.maximum((h_ref[...] - mean) * scale + bet_ref[...], 0.0)
    hw = jnp.dot(y, w_ref[...], preferred_element_type=jnp.float32)
    g_ref[...] = dinv_ref[...] * hw


def _tc_bn_mm(h, stats, gamma, beta, w2, dinv):
    return pl.pallas_call(
        _tc_bn_mm_body,
        grid=(_GRID,),
        in_specs=[
            pl.BlockSpec((_BLK, D), lambda i: (i, 0)),
            pl.BlockSpec((8, D), lambda i: (0, 0)),
            pl.BlockSpec((1, D), lambda i: (0, 0)),
            pl.BlockSpec((1, D), lambda i: (0, 0)),
            pl.BlockSpec((D, D), lambda i: (0, 0)),
            pl.BlockSpec((_BLK, 1), lambda i: (i, 0)),
        ],
        out_specs=pl.BlockSpec((_BLK, D), lambda i: (i, 0)),
        out_shape=jax.ShapeDtypeStruct((N_PAD, D), jnp.float32),
    )(h, stats, gamma, beta, w2, dinv)


def _tc_combine_body(p_ref, g_ref, dinv_ref, b_ref, out_ref):
    out_ref[...] = (
        dinv_ref[...] * (p_ref[0] + p_ref[1] + g_ref[...]) + b_ref[...]
    )


def _tc_combine(parts, g, dinv, b):
    return pl.pallas_call(
        _tc_combine_body,
        grid=(_GRID,),
        in_specs=_COMBINE_SPECS,
        out_specs=pl.BlockSpec((_BLK, D), lambda i: (i, 0)),
        out_shape=jax.ShapeDtypeStruct((N_PAD, D), jnp.float32),
    )(parts, g, dinv, b)


@jax.jit
def kernel(x, edge_index, W1, b1, gamma, beta, W2, b2):
    e = edge_index.shape[1]
    pad = EP - e
    src = jnp.concatenate(
        [edge_index[0].astype(jnp.int32), jnp.full((pad,), N, jnp.int32)]
    ).reshape(NW, CH, CS)
    dst = jnp.concatenate(
        [edge_index[1].astype(jnp.int32), jnp.full((pad,), N, jnp.int32)]
    ).reshape(NW, CH, CS)
    x_pad = jnp.zeros((N_PAD, D), jnp.float32).at[:N].set(x)

    deg_parts = _sc_deg(dst)
    g1, dinv = _tc_prep(deg_parts.reshape(NC, N_PAD, 1), x_pad, W1)

    s_parts = _sc_scatter(g1, src, dst)
    h1, stats = _tc_combine_stats(s_parts, g1, dinv, b1.reshape(1, D))

    g2 = _tc_bn_mm(h1, stats, gamma.reshape(1, D), beta.reshape(1, D),
                   W2, dinv)
    t_parts = _sc_scatter(g2, src, dst)
    out_full = _tc_combine(t_parts, g2, dinv, b2.reshape(1, D))
    return out_full[:N]
